# baseline (speedup 1.0000x reference)
import functools

import jax
import jax.numpy as jnp
from jax import lax
from jax.experimental import pallas as pl
from jax.experimental.pallas import tpu as pltpu
from jax.experimental.pallas import tpu_sc as plsc

VOCAB = 1000000
EMB_DIM = 64
BATCH = 16384

_INFO = plsc.get_sparse_core_info()
_NC, _NS = _INFO.num_cores, _INFO.num_subcores
_NW = _NC * _NS
_B_PER_W = BATCH // _NW
_CHUNK = 128
_NCHUNK = _B_PER_W // _CHUNK

_mesh = plsc.VectorSubcoreMesh(core_axis_name="c", subcore_axis_name="s")


@functools.partial(
    pl.kernel,
    out_type=jax.ShapeDtypeStruct((BATCH, EMB_DIM), jnp.float32),
    mesh=_mesh,
    scratch_types=[
        pltpu.VMEM((_NCHUNK, _CHUNK), jnp.int32),
        pltpu.VMEM((_B_PER_W, EMB_DIM), jnp.float32),
        pltpu.SemaphoreType.DMA,
    ],
    compiler_params=pltpu.CompilerParams(use_tc_tiling_on_sc=False),
)
def _gather_kernel(idx_hbm, table_hbm, out_hbm, idx_v, rows_v, sem):
    wid = lax.axis_index("s") * _NC + lax.axis_index("c")
    base = wid * _B_PER_W
    pltpu.sync_copy(idx_hbm.at[wid], idx_v)
    copies = []
    for j in range(_NCHUNK):
        copies.append(
            pltpu.async_copy(
                table_hbm.at[idx_v.at[j]],
                rows_v.at[pl.ds(j * _CHUNK, _CHUNK)],
                sem,
            )
        )
    for c in copies:
        c.wait()
    pltpu.sync_copy(rows_v, out_hbm.at[pl.ds(base, _B_PER_W)])


def kernel(node_idx, emb):
    idx = node_idx.astype(jnp.int32).reshape(_NW, _NCHUNK, _CHUNK)
    return _gather_kernel(idx, emb)


# speedup vs baseline: 1.7883x; 1.7883x over previous
import functools

import jax
import jax.numpy as jnp
from jax import lax
from jax.experimental import pallas as pl
from jax.experimental.pallas import tpu as pltpu
from jax.experimental.pallas import tpu_sc as plsc

VOCAB = 1000000
EMB_DIM = 64
BATCH = 16384

_NW = 32
_CW = 512
_MAIN_V = 999936
_NCHUNKS = _MAIN_V // _CW
_TAIL_V = VOCAB - _MAIN_V
_NL = 62
_TAIL_L = 62
_CAP = 768
_RING = 64
_SENT = jnp.int32(0x7FFFFFFF)

_mesh = plsc.VectorSubcoreMesh(core_axis_name="c", subcore_axis_name="s")


def _worker_id():
    return lax.axis_index("s") * 2 + lax.axis_index("c")


@functools.partial(
    pl.kernel,
    out_type=(
        jax.ShapeDtypeStruct(((_NW * _CAP + 16) * EMB_DIM,), jnp.float32),
        jax.ShapeDtypeStruct((_NW * _CAP,), jnp.int32),
        jax.ShapeDtypeStruct((_NW * 16,), jnp.int32),
    ),
    mesh=_mesh,
    scratch_types=[
        pltpu.VMEM((4096,), jnp.int32),
        pltpu.VMEM((_CAP + 16,), jnp.int32),
        pltpu.VMEM((_CAP + 16,), jnp.int32),
        pltpu.VMEM((_CAP + 16,), jnp.int32),
        pltpu.VMEM((2, EMB_DIM, _CW), jnp.float32),
        pltpu.VMEM((EMB_DIM, _TAIL_V), jnp.float32),
        pltpu.VMEM((_RING * EMB_DIM,), jnp.float32),
        pltpu.VMEM((_CAP + 16,), jnp.int32),
        pltpu.VMEM((16,), jnp.int32),
        pltpu.SemaphoreType.DMA,
        pltpu.SemaphoreType.DMA,
    ],
    compiler_params=pltpu.CompilerParams(needs_layout_passes=False),
)
def _phase1(idx_hbm, table_hbm, rows_hbm, p_hbm, cnt_hbm,
            idxbuf, hp_v, sb_v, clist, chunkbuf, tailbuf, ring, pv, cvec,
            sem_ch, sem_rows):
    w = _worker_id()
    lane = lax.iota(jnp.int32, 16)

    def extract(ref, pos):
        vv = ref[pl.ds((pos >> 4) * 16, 16)]
        return jnp.sum(jnp.where(lane == (pos & 15), vv, 0))

    v0 = pl.multiple_of(w * _CW, _CW)
    pltpu.async_copy(table_hbm.at[:, pl.ds(v0, _CW)], chunkbuf.at[0], sem_ch)

    off = jnp.int32(0)
    for b in range(4):
        pltpu.sync_copy(idx_hbm.at[pl.ds(b * 4096, 4096)], idxbuf)

        def eb(g, off, b=b):
            vec = idxbuf[pl.ds(g * 16, 16)]
            ist = vec >= _MAIN_V
            owner = jnp.where(ist, 31, lax.shift_right_logical(vec, 9) & 31)
            m = owner == w
            lidx = jnp.where(ist, _TAIL_L, lax.shift_right_logical(vec, 14))
            u = jnp.where(ist, vec - _MAIN_V, vec & (_CW - 1))
            ipos = lane + (b * 4096) + g * 16
            e = (lidx << 23) | (u << 14) | ipos
            _, ev, _ = plsc.sort_key_val(e, e, mask=m)
            hp_v[pl.ds(off, 16)] = ev
            return off + jnp.sum(m.astype(jnp.int32))

        off = lax.fori_loop(0, 256, eb, off)

    hp_v[pl.ds(off, 16)] = jnp.full((16,), _SENT)
    nv = lax.shift_right_logical(off + 15, 4)

    sboff = jnp.int32(0)
    endvec = jnp.zeros((16,), jnp.int32)
    for s in range(8):
        def sb_body(g, sboff, s=s):
            vec = hp_v[pl.ds(g * 16, 16)]
            m = lax.shift_right_logical(vec, 26) == s
            _, sv, _ = plsc.sort_key_val(vec, vec, mask=m)
            sb_v[pl.ds(sboff, 16)] = sv
            return sboff + jnp.sum(m.astype(jnp.int32))

        sboff = lax.fori_loop(0, nv, sb_body, sboff)
        endvec = jnp.where(lane == s, sboff, endvec)
    sb_v[pl.ds(sboff, 16)] = jnp.full((16,), _SENT)

    rowbase = w * _CAP

    def make_hit(buf):
        def hit(k, carry, buf=buf):
            hcur, coff = carry
            e = extract(clist, k)
            u = lax.shift_right_logical(e, 14) & (_CW - 1)
            i = e & (BATCH - 1)
            pv[pl.ds(hcur, 16)] = jnp.full((16,), i, jnp.int32)
            slot = hcur & (_RING - 1)
            colv = jnp.full((16,), u, jnp.int32)
            for q in range(4):
                vecq = plsc.load_gather(buf, [lane + 16 * q, colv])
                ring[pl.ds(slot * EMB_DIM + q * 16, 16)] = vecq
            pltpu.async_copy(
                ring.at[pl.ds(slot * EMB_DIM, EMB_DIM)],
                rows_hbm.at[pl.ds((rowbase + hcur) * EMB_DIM, EMB_DIM)],
                sem_rows)

            @pl.when(hcur >= _RING)
            def _():
                pltpu.make_async_copy(rows_hbm.at[pl.ds(0, EMB_DIM)],
                                      ring.at[pl.ds(0, EMB_DIM)],
                                      sem_rows).wait()
            return (hcur + 1, coff)
        return hit

    def bucket_hits(l, hcur, buf):
        s = lax.shift_right_logical(l, 3)
        lo = jnp.sum(jnp.where(lane == s - 1, endvec, 0))
        hi = jnp.sum(jnp.where(lane == s, endvec, 0))

        def scan_body(g, coff):
            vec = sb_v[pl.ds(g * 16, 16)]
            m = lax.shift_right_logical(vec, 23) == l
            _, sv, _ = plsc.sort_key_val(vec, vec, mask=m)
            clist[pl.ds(coff, 16)] = sv
            return coff + jnp.sum(m.astype(jnp.int32))

        coff = lax.fori_loop(lo >> 4, lax.shift_right_logical(hi + 15, 4),
                             scan_body, jnp.int32(0))
        hcur, _ = lax.fori_loop(0, coff, make_hit(buf), (hcur, coff))
        return hcur

    def chunk_body(l, hcur):
        c = l * _NW + w

        def run(hcur):
            par = l & 1
            pltpu.make_async_copy(table_hbm.at[:, pl.ds(0, _CW)],
                                  chunkbuf.at[par], sem_ch).wait()
            cn = c + _NW

            @pl.when(cn < _NCHUNKS)
            def _():
                vn = pl.multiple_of(cn * _CW, _CW)
                pltpu.async_copy(table_hbm.at[:, pl.ds(vn, _CW)],
                                 chunkbuf.at[1 - par], sem_ch)

            return bucket_hits(l, hcur, chunkbuf.at[par])

        return lax.cond(c < _NCHUNKS, run, lambda h: h, hcur)

    hcur = lax.fori_loop(0, _NL, chunk_body, jnp.int32(0))

    def run_tail(hcur):
        pltpu.sync_copy(table_hbm.at[:, pl.ds(_MAIN_V, _TAIL_V)], tailbuf)
        return bucket_hits(jnp.int32(_TAIL_L), hcur, tailbuf)

    hcur = lax.cond(w == _NW - 1, run_tail, lambda h: h, hcur)

    def dr(k, _):
        pltpu.make_async_copy(rows_hbm.at[pl.ds(0, EMB_DIM)],
                              ring.at[pl.ds(0, EMB_DIM)], sem_rows).wait()
        return 0
    lax.fori_loop(0, jnp.minimum(hcur, _RING), dr, 0)

    pltpu.sync_copy(pv.at[pl.ds(0, _CAP)], p_hbm.at[pl.ds(w * _CAP, _CAP)])
    cvec[pl.ds(0, 16)] = jnp.full((16,), off, jnp.int32)
    pltpu.sync_copy(cvec, cnt_hbm.at[pl.ds(w * 16, 16)])


@functools.partial(
    pl.kernel,
    out_type=jax.ShapeDtypeStruct((BATCH + 8, EMB_DIM), jnp.float32),
    mesh=_mesh,
    scratch_types=[
        pltpu.VMEM((_CAP, EMB_DIM), jnp.float32),
        pltpu.VMEM((_CAP // 128, 128), jnp.int32),
        pltpu.VMEM((16,), jnp.int32),
        pltpu.SemaphoreType.DMA,
    ],
    compiler_params=pltpu.CompilerParams(use_tc_tiling_on_sc=False,
                                         needs_layout_passes=False),
)
def _phase2(rows_hbm, p_hbm, cnt_hbm, out_hbm, rows_v, pv2, cvec2, sem):
    w = _worker_id()
    lane = lax.iota(jnp.int32, 16)
    pltpu.sync_copy(cnt_hbm.at[pl.ds(w * 16, 16)], cvec2)
    pltpu.sync_copy(rows_hbm.at[pl.ds(w * _CAP, _CAP)], rows_v)
    pltpu.sync_copy(p_hbm.at[w], pv2)
    cnt_vec = cvec2[pl.ds(0, 16)]
    for k in range(_CAP // 128):
        for j in range(8):
            vec = pv2[k, pl.ds(j * 16, 16)]
            m = (k * 128 + j * 16 + lane) < cnt_vec
            pv2[k, pl.ds(j * 16, 16)] = jnp.where(m, vec, BATCH)
    copies = []
    for k in range(_CAP // 128):
        copies.append(
            pltpu.async_copy(rows_v.at[pl.ds(k * 128, 128)],
                             out_hbm.at[pv2.at[k]], sem))
    for cp in copies:
        cp.wait()


def kernel(node_idx, emb):
    idx = node_idx.astype(jnp.int32)
    rows, p, cnt = _phase1(idx, emb.T)
    rows2 = rows.reshape(_NW * _CAP + 16, EMB_DIM)
    p3 = p.reshape(_NW, _CAP // 128, 128)
    outp = _phase2(rows2, p3, cnt)
    return outp[:BATCH]


# speedup vs baseline: 3.4383x; 1.9227x over previous
import functools

import jax
import jax.numpy as jnp
from jax import lax
from jax.experimental import pallas as pl
from jax.experimental.pallas import tpu as pltpu
from jax.experimental.pallas import tpu_sc as plsc

VOCAB = 1000000
EMB_DIM = 64
BATCH = 16384

_NW = 32
_CW = 512
_MAIN_V = 999936
_NCHUNKS = _MAIN_V // _CW
_TAIL_V = VOCAB - _MAIN_V
_NL = 62
_TAIL_L = 62
_CAP = 768
_RING = 64
_SENT = jnp.int32(0x7FFFFFFF)

_mesh = plsc.VectorSubcoreMesh(core_axis_name="c", subcore_axis_name="s")


@functools.partial(
    pl.kernel,
    out_type=jax.ShapeDtypeStruct((BATCH * EMB_DIM,), jnp.float32),
    mesh=_mesh,
    scratch_types=[
        pltpu.VMEM((4096,), jnp.int32),
        pltpu.VMEM((_CAP + 16,), jnp.int32),
        pltpu.VMEM((_CAP + 16,), jnp.int32),
        pltpu.VMEM((_CAP + 16,), jnp.int32),
        pltpu.VMEM((2, EMB_DIM, _CW), jnp.float32),
        pltpu.VMEM((EMB_DIM, _TAIL_V), jnp.float32),
        pltpu.VMEM((_RING * EMB_DIM,), jnp.float32),
        pltpu.SemaphoreType.DMA,
        pltpu.SemaphoreType.DMA,
    ],
    compiler_params=pltpu.CompilerParams(needs_layout_passes=False),
)
def _gather(idx_hbm, table_hbm, out_hbm,
            idxbuf, hp_v, sb_v, clist, chunkbuf, tailbuf, ring,
            sem_ch, sem_rows):
    w = lax.axis_index("s") * 2 + lax.axis_index("c")
    lane = lax.iota(jnp.int32, 16)

    def extract(ref, pos):
        vv = ref[pl.ds((pos >> 4) * 16, 16)]
        return jnp.sum(jnp.where(lane == (pos & 15), vv, 0))

    v0 = pl.multiple_of(w * _CW, _CW)
    pltpu.async_copy(table_hbm.at[:, pl.ds(v0, _CW)], chunkbuf.at[0], sem_ch)

    off = jnp.int32(0)
    for b in range(4):
        pltpu.sync_copy(idx_hbm.at[pl.ds(b * 4096, 4096)], idxbuf)

        def eb(g, off, b=b):
            vec = idxbuf[pl.ds(g * 16, 16)]
            ist = vec >= _MAIN_V
            owner = jnp.where(ist, 31, lax.shift_right_logical(vec, 9) & 31)
            m = owner == w
            lidx = jnp.where(ist, _TAIL_L, lax.shift_right_logical(vec, 14))
            u = jnp.where(ist, vec - _MAIN_V, vec & (_CW - 1))
            ipos = lane + (b * 4096) + g * 16
            e = (lidx << 23) | (u << 14) | ipos
            _, ev, _ = plsc.sort_key_val(e, e, mask=m)
            hp_v[pl.ds(off, 16)] = ev
            return off + jnp.sum(m.astype(jnp.int32))

        off = lax.fori_loop(0, 256, eb, off)

    hp_v[pl.ds(off, 16)] = jnp.full((16,), _SENT)
    nv = lax.shift_right_logical(off + 15, 4)

    sboff = jnp.int32(0)
    endvec = jnp.zeros((16,), jnp.int32)
    for s in range(8):
        def sb_body(g, sboff, s=s):
            vec = hp_v[pl.ds(g * 16, 16)]
            m = lax.shift_right_logical(vec, 26) == s
            _, sv, _ = plsc.sort_key_val(vec, vec, mask=m)
            sb_v[pl.ds(sboff, 16)] = sv
            return sboff + jnp.sum(m.astype(jnp.int32))

        sboff = lax.fori_loop(0, nv, sb_body, sboff)
        endvec = jnp.where(lane == s, sboff, endvec)
    sb_v[pl.ds(sboff, 16)] = jnp.full((16,), _SENT)

    def make_hit(buf):
        def hit(k, carry, buf=buf):
            hcur, _c = carry
            e = extract(clist, k)
            u = lax.shift_right_logical(e, 14) & (_CW - 1)
            i = e & (BATCH - 1)
            slot = hcur & (_RING - 1)
            colv = jnp.full((16,), u, jnp.int32)
            for q in range(4):
                vecq = plsc.load_gather(buf, [lane + 16 * q, colv])
                ring[pl.ds(slot * EMB_DIM + q * 16, 16)] = vecq
            pltpu.async_copy(
                ring.at[pl.ds(slot * EMB_DIM, EMB_DIM)],
                out_hbm.at[pl.ds(i * EMB_DIM, EMB_DIM)],
                sem_rows)

            @pl.when(hcur >= _RING)
            def _():
                pltpu.make_async_copy(out_hbm.at[pl.ds(0, EMB_DIM)],
                                      ring.at[pl.ds(0, EMB_DIM)],
                                      sem_rows).wait()
            return (hcur + 1, _c)
        return hit

    def bucket_hits(l, hcur, buf):
        s = lax.shift_right_logical(l, 3)
        lo = jnp.sum(jnp.where(lane == s - 1, endvec, 0))
        hi = jnp.sum(jnp.where(lane == s, endvec, 0))

        def scan_body(g, coff):
            vec = sb_v[pl.ds(g * 16, 16)]
            m = lax.shift_right_logical(vec, 23) == l
            _, sv, _ = plsc.sort_key_val(vec, vec, mask=m)
            clist[pl.ds(coff, 16)] = sv
            return coff + jnp.sum(m.astype(jnp.int32))

        coff = lax.fori_loop(lo >> 4, lax.shift_right_logical(hi + 15, 4),
                             scan_body, jnp.int32(0))
        hcur, _ = lax.fori_loop(0, coff, make_hit(buf), (hcur, coff))
        return hcur

    def chunk_body(l, hcur):
        c = l * _NW + w

        def run(hcur):
            par = l & 1
            pltpu.make_async_copy(table_hbm.at[:, pl.ds(0, _CW)],
                                  chunkbuf.at[par], sem_ch).wait()
            cn = c + _NW

            @pl.when(cn < _NCHUNKS)
            def _():
                vn = pl.multiple_of(cn * _CW, _CW)
                pltpu.async_copy(table_hbm.at[:, pl.ds(vn, _CW)],
                                 chunkbuf.at[1 - par], sem_ch)

            return bucket_hits(l, hcur, chunkbuf.at[par])

        return lax.cond(c < _NCHUNKS, run, lambda h: h, hcur)

    hcur = lax.fori_loop(0, _NL, chunk_body, jnp.int32(0))

    def run_tail(hcur):
        pltpu.sync_copy(table_hbm.at[:, pl.ds(_MAIN_V, _TAIL_V)], tailbuf)
        return bucket_hits(jnp.int32(_TAIL_L), hcur, tailbuf)

    hcur = lax.cond(w == _NW - 1, run_tail, lambda h: h, hcur)

    def dr(k, _):
        pltpu.make_async_copy(out_hbm.at[pl.ds(0, EMB_DIM)],
                              ring.at[pl.ds(0, EMB_DIM)], sem_rows).wait()
        return 0
    lax.fori_loop(0, jnp.minimum(hcur, _RING), dr, 0)


def kernel(node_idx, emb):
    idx = node_idx.astype(jnp.int32)
    flat = _gather(idx, emb.T)
    return flat.reshape(BATCH, EMB_DIM)
